# baseline (speedup 1.0000x reference)
import functools

import jax
import jax.numpy as jnp
import numpy as np
from jax import lax
from jax.experimental import pallas as pl
from jax.experimental.pallas import tpu as pltpu
from jax.experimental.pallas import tpu_sc as plsc

B = 16384
D = 128
DW = D // 2
NC = 2
NS = 16
NW = NC * NS
CH = 128
NBUF = 2

_PERM = np.empty((D,), dtype=np.int32)
for _q in range(4):
    for _t in range(16):
        _PERM[32 * _q + 2 * _t] = 32 * _q + _t
        _PERM[32 * _q + 2 * _t + 1] = 32 * _q + 16 + _t


@functools.cache
def _build_sc_gather():
    bpw = B // NW
    nch = bpw // CH
    mesh = plsc.VectorSubcoreMesh(core_axis_name="c", subcore_axis_name="s")

    @functools.partial(
        pl.kernel,
        mesh=mesh,
        out_type=(
            jax.ShapeDtypeStruct((B, DW), jnp.int32),
            jax.ShapeDtypeStruct((B, DW), jnp.int32),
        ),
        scratch_types=[
            pltpu.VMEM((bpw,), jnp.int32),
            pltpu.VMEM((bpw,), jnp.int32),
            pltpu.VMEM((NBUF, CH, D), jnp.float32),
            pltpu.VMEM((NBUF, CH, D), jnp.float32),
            pltpu.VMEM((CH, DW), jnp.int32),
            pltpu.VMEM((CH, DW), jnp.int32),
            pltpu.SemaphoreType.DMA,
            pltpu.SemaphoreType.DMA,
            pltpu.SemaphoreType.DMA,
            pltpu.SemaphoreType.DMA,
            pltpu.SemaphoreType.DMA,
        ],
    )
    def _sc_gather(gidx_hbm, iidx_hbm, gtab_hbm, itab_hbm, gout_hbm, iout_hbm,
                   gidx_v, iidx_v, gbuf, ibuf, gcv, icv, sem_x, sem_g, sem_i,
                   sem_wg, sem_wi):
        wid = lax.axis_index("s") * NC + lax.axis_index("c")
        base = wid * bpw
        cx = pltpu.async_copy(gidx_hbm.at[pl.ds(base, bpw)], gidx_v, sem_x)
        cy = pltpu.async_copy(iidx_hbm.at[pl.ds(base, bpw)], iidx_v, sem_x)
        cx.wait()
        cy.wait()

        def fire_g(c):
            return pltpu.async_copy(
                gtab_hbm.at[gidx_v.at[pl.ds(c * CH, CH)]], gbuf.at[c % NBUF],
                sem_g)

        def fire_i(c):
            return pltpu.async_copy(
                itab_hbm.at[iidx_v.at[pl.ds(c * CH, CH)]], ibuf.at[c % NBUF],
                sem_i)

        def convert(src, dst, s):
            @plsc.parallel_loop(0, CH, unroll=8)
            def _row(r):
                for q in range(4):
                    a = src[s, r, pl.ds(32 * q, 16)]
                    b = src[s, r, pl.ds(32 * q + 16, 16)]
                    ua = lax.bitcast_convert_type(a, jnp.uint32)
                    ub = lax.bitcast_convert_type(b, jnp.uint32)
                    w = (ua >> jnp.uint32(16)) | (ub & jnp.uint32(0xFFFF0000))
                    dst[r, pl.ds(16 * q, 16)] = lax.bitcast_convert_type(
                        w, jnp.int32)

        gc = [None] * nch
        ic = [None] * nch
        gw = [None] * nch
        iw = [None] * nch
        for c in range(NBUF):
            gc[c] = fire_g(c)
            ic[c] = fire_i(c)
        for c in range(nch):
            s = c % NBUF
            off = base + c * CH
            gc[c].wait()
            if c >= 1:
                gw[c - 1].wait()
            convert(gbuf, gcv, s)
            if c + NBUF < nch:
                gc[c + NBUF] = fire_g(c + NBUF)
            gw[c] = pltpu.async_copy(gcv, gout_hbm.at[pl.ds(off, CH)], sem_wg)
            ic[c].wait()
            if c >= 1:
                iw[c - 1].wait()
            convert(ibuf, icv, s)
            if c + NBUF < nch:
                ic[c + NBUF] = fire_i(c + NBUF)
            iw[c] = pltpu.async_copy(icv, iout_hbm.at[pl.ds(off, CH)], sem_wi)
        gw[nch - 1].wait()
        iw[nch - 1].wait()

    return _sc_gather


BT = 8192


def _mlp_body(g_ref, i_ref, a_ref, b_ref, c_ref, b1_ref, w2_ref, b2_ref, o_ref):
    g = g_ref[...]
    it = i_ref[...]
    m = g * it
    h = (jnp.dot(m, a_ref[...], preferred_element_type=jnp.float32)
         + jnp.dot(g, b_ref[...], preferred_element_type=jnp.float32)
         + jnp.dot(it, c_ref[...], preferred_element_type=jnp.float32)
         + b1_ref[...])
    h = jnp.maximum(h, 0.0)
    y = jnp.sum(h * w2_ref[...], axis=1, keepdims=True) + b2_ref[...]
    o_ref[...] = 1.0 / (1.0 + jnp.exp(-y))


def _mlp(g_rows, i_rows, W1a, W1b, W1c, b1, W2, b2):
    n = g_rows.shape[0]
    bt = min(BT, n)
    return pl.pallas_call(
        _mlp_body,
        grid=(n // bt,),
        in_specs=[
            pl.BlockSpec((bt, D), lambda b: (b, 0)),
            pl.BlockSpec((bt, D), lambda b: (b, 0)),
            pl.BlockSpec((D, 8), lambda b: (0, 0)),
            pl.BlockSpec((D, 8), lambda b: (0, 0)),
            pl.BlockSpec((D, 8), lambda b: (0, 0)),
            pl.BlockSpec((1, 8), lambda b: (0, 0)),
            pl.BlockSpec((1, 8), lambda b: (0, 0)),
            pl.BlockSpec((1, 1), lambda b: (0, 0)),
        ],
        out_specs=pl.BlockSpec((bt, 1), lambda b: (b, 0)),
        out_shape=jax.ShapeDtypeStruct((n, 1), jnp.float32),
    )(g_rows, i_rows, W1a, W1b, W1c, b1, W2, b2)


def kernel(group_inputs, item_inputs, group_table, item_table, W1, b1, W2, b2):
    gidx = group_inputs.astype(jnp.int32)
    iidx = item_inputs.astype(jnp.int32)
    perm = jnp.asarray(_PERM)
    W1a = W1[0:D][perm].astype(jnp.bfloat16)
    W1b = W1[D:2 * D][perm].astype(jnp.bfloat16)
    W1c = W1[2 * D:3 * D][perm].astype(jnp.bfloat16)
    b1r = b1.reshape(1, 8)
    w2r = W2.reshape(1, 8)
    b2r = b2.reshape(1, 1)
    g_pk, i_pk = _build_sc_gather()(gidx, iidx, group_table, item_table)
    g16 = lax.bitcast_convert_type(g_pk, jnp.bfloat16).reshape(B, D)
    i16 = lax.bitcast_convert_type(i_pk, jnp.bfloat16).reshape(B, D)
    return _mlp(g16, i16, W1a, W1b, W1c, b1r, w2r, b2r)


# speedup vs baseline: 2.6694x; 2.6694x over previous
import functools

import jax
import jax.numpy as jnp
from jax import lax
from jax.experimental import pallas as pl
from jax.experimental.pallas import tpu as pltpu
from jax.experimental.pallas import tpu_sc as plsc

B = 16384
D = 128
NC = 2
NS = 16
NW = NC * NS
CH = 128
NBUF = 3


@functools.cache
def _build_sc_gather(nrows):
    bpw = nrows // NW
    nch = bpw // CH
    nbuf = min(NBUF, nch)
    mesh = plsc.VectorSubcoreMesh(core_axis_name="c", subcore_axis_name="s")

    @functools.partial(
        pl.kernel,
        mesh=mesh,
        out_type=(
            jax.ShapeDtypeStruct((nrows, D), jnp.float32),
            jax.ShapeDtypeStruct((nrows, D), jnp.float32),
        ),
        scratch_types=[
            pltpu.VMEM((bpw,), jnp.int32),
            pltpu.VMEM((bpw,), jnp.int32),
            pltpu.VMEM((nbuf, CH, D), jnp.float32),
            pltpu.VMEM((nbuf, CH, D), jnp.float32),
            pltpu.SemaphoreType.DMA,
            pltpu.SemaphoreType.DMA,
            pltpu.SemaphoreType.DMA,
            pltpu.SemaphoreType.DMA,
            pltpu.SemaphoreType.DMA,
        ],
    )
    def _sc_gather(gidx_hbm, iidx_hbm, gtab_hbm, itab_hbm, gout_hbm, iout_hbm,
                   gidx_v, iidx_v, gbuf, ibuf, sem_x, sem_g, sem_i, sem_wg,
                   sem_wi):
        wid = lax.axis_index("s") * NC + lax.axis_index("c")
        base = wid * bpw
        cx = pltpu.async_copy(gidx_hbm.at[pl.ds(base, bpw)], gidx_v, sem_x)
        cy = pltpu.async_copy(iidx_hbm.at[pl.ds(base, bpw)], iidx_v, sem_x)
        cx.wait()
        cy.wait()

        def fire_g(c):
            return pltpu.async_copy(
                gtab_hbm.at[gidx_v.at[pl.ds(c * CH, CH)]], gbuf.at[c % nbuf],
                sem_g)

        def fire_i(c):
            return pltpu.async_copy(
                itab_hbm.at[iidx_v.at[pl.ds(c * CH, CH)]], ibuf.at[c % nbuf],
                sem_i)

        gc = [None] * nch
        ic = [None] * nch
        gw = [None] * nch
        iw = [None] * nch
        for c in range(nbuf):
            gc[c] = fire_g(c)
            ic[c] = fire_i(c)
        for c in range(nch):
            off = base + c * CH
            gc[c].wait()
            gw[c] = pltpu.async_copy(gbuf.at[c % nbuf],
                                     gout_hbm.at[pl.ds(off, CH)], sem_wg)
            ic[c].wait()
            iw[c] = pltpu.async_copy(ibuf.at[c % nbuf],
                                     iout_hbm.at[pl.ds(off, CH)], sem_wi)
            fc = c + nbuf
            if fc < nch:
                gw[c].wait()
                gc[fc] = fire_g(fc)
                iw[c].wait()
                ic[fc] = fire_i(fc)
        for c in range(nch):
            if c + nbuf >= nch:
                gw[c].wait()
                iw[c].wait()

    return _sc_gather


BT = 8192


def _mlp_body(g_ref, i_ref, a_ref, b_ref, c_ref, b1_ref, w2_ref, b2_ref, o_ref):
    g = g_ref[...]
    it = i_ref[...]
    m = g * it
    h = (jnp.dot(m, a_ref[...], preferred_element_type=jnp.float32)
         + jnp.dot(g, b_ref[...], preferred_element_type=jnp.float32)
         + jnp.dot(it, c_ref[...], preferred_element_type=jnp.float32)
         + b1_ref[...])
    h = jnp.maximum(h, 0.0)
    y = jnp.sum(h * w2_ref[...], axis=1, keepdims=True) + b2_ref[...]
    o_ref[...] = 1.0 / (1.0 + jnp.exp(-y))


def _mlp(g_rows, i_rows, W1a, W1b, W1c, b1, W2, b2):
    n = g_rows.shape[0]
    bt = min(BT, n)
    return pl.pallas_call(
        _mlp_body,
        grid=(n // bt,),
        in_specs=[
            pl.BlockSpec((bt, D), lambda b: (b, 0)),
            pl.BlockSpec((bt, D), lambda b: (b, 0)),
            pl.BlockSpec((D, 8), lambda b: (0, 0)),
            pl.BlockSpec((D, 8), lambda b: (0, 0)),
            pl.BlockSpec((D, 8), lambda b: (0, 0)),
            pl.BlockSpec((1, 8), lambda b: (0, 0)),
            pl.BlockSpec((1, 8), lambda b: (0, 0)),
            pl.BlockSpec((1, 1), lambda b: (0, 0)),
        ],
        out_specs=pl.BlockSpec((bt, 1), lambda b: (b, 0)),
        out_shape=jax.ShapeDtypeStruct((n, 1), jnp.float32),
    )(g_rows, i_rows, W1a, W1b, W1c, b1, W2, b2)


def kernel(group_inputs, item_inputs, group_table, item_table, W1, b1, W2, b2):
    gidx = group_inputs.astype(jnp.int32)
    iidx = item_inputs.astype(jnp.int32)
    W1a = W1[0:D]
    W1b = W1[D:2 * D]
    W1c = W1[2 * D:3 * D]
    b1r = b1.reshape(1, 8)
    w2r = W2.reshape(1, 8)
    b2r = b2.reshape(1, 1)
    g_rows, i_rows = _build_sc_gather(B)(gidx, iidx, group_table, item_table)
    return _mlp(g_rows, i_rows, W1a, W1b, W1c, b1r, w2r, b2r)
